# baseline (speedup 1.0000x reference)
import functools
import jax
import jax.numpy as jnp
from jax import lax
from jax.experimental import pallas as pl
from jax.experimental.pallas import tpu as pltpu
from jax.experimental.pallas import tpu_sc as plsc

B, D, O, E = 8192, 768, 768, 8
NBUK = 64
GP = 128
CAPP = B + 28 * GP
NBLK = CAPP // GP
NC, NS, L = 2, 16, 16
NW = NC * NS
TPT = B // NW
TBG = 512
XCH = 64


def _gate_body(x_ref, wg_ref, bg_ref, pid_ref, wa_ref, wb_ref, cnt_ref):
    x = x_ref[...]
    logits = lax.dot_general(
        x, wg_ref[...], (((1,), (0,)), ((), ())),
        preferred_element_type=jnp.float32,
    ) + bg_ref[...][None, :]
    col = lax.broadcasted_iota(jnp.int32, (TBG, E), 1)
    m0 = jnp.max(logits, axis=1, keepdims=True)
    i0 = jnp.min(jnp.where(logits == m0, col, E), axis=1, keepdims=True)
    masked = jnp.where(col == i0, -jnp.inf, logits)
    m1 = jnp.max(masked, axis=1, keepdims=True)
    i1 = jnp.min(jnp.where(masked == m1, col, E), axis=1, keepdims=True)
    d = jnp.exp(m1 - m0)
    p0 = 1.0 / (1.0 + d)
    p1 = d / (1.0 + d)
    i0s, i1s = i0[:, 0], i1[:, 0]
    p0s, p1s = p0[:, 0], p1[:, 0]
    a = jnp.minimum(i0s, i1s)
    bmx = jnp.maximum(i0s, i1s)
    pid = a * E + bmx
    pid_ref[...] = pid
    first_is_a = i0s < i1s
    wa_ref[...] = jnp.where(first_is_a, p0s, p1s)
    wb_ref[...] = jnp.where(first_is_a, p1s, p0s)
    buk = lax.broadcasted_iota(jnp.int32, (TBG, NBUK), 1)
    oh = (pid[:, None] == buk).astype(jnp.int32)
    half = lax.broadcasted_iota(jnp.int32, (TBG, NBUK), 0) < (TBG // 2)
    cnt_ref[0, 0, :] = jnp.sum(jnp.where(half, oh, 0), axis=0)
    cnt_ref[0, 1, :] = jnp.sum(jnp.where(half, 0, oh), axis=0)


def _gate(x, Wg, bg):
    return pl.pallas_call(
        _gate_body,
        grid=(B // TBG,),
        in_specs=[
            pl.BlockSpec((TBG, D), lambda i: (i, 0)),
            pl.BlockSpec((D, E), lambda i: (0, 0)),
            pl.BlockSpec((E,), lambda i: (0,)),
        ],
        out_specs=[
            pl.BlockSpec((TBG,), lambda i: (i,)),
            pl.BlockSpec((TBG,), lambda i: (i,)),
            pl.BlockSpec((TBG,), lambda i: (i,)),
            pl.BlockSpec((1, 2, NBUK), lambda i: (i, 0, 0)),
        ],
        out_shape=[
            jax.ShapeDtypeStruct((B,), jnp.int32),
            jax.ShapeDtypeStruct((B,), jnp.float32),
            jax.ShapeDtypeStruct((B,), jnp.float32),
            jax.ShapeDtypeStruct((B // TBG, 2, NBUK), jnp.int32),
        ],
    )(x, Wg, bg)


def _route_body(x_hbm, pid_hbm, wa_hbm, wb_hbm, start_hbm,
                xg_hbm, swa_hbm, swb_hbm, pos_hbm,
                pid_v, wa_v, wb_v, start_v, cnt_v, pos_v,
                xrow_v, idx_v, was_v, wbs_v, tmp_v, sem):
    c = lax.axis_index("c")
    s = lax.axis_index("s")
    wid = s * NC + c
    base = wid * TPT
    iota = lax.iota(jnp.int32, L)

    pltpu.sync_copy(pid_hbm.at[pl.ds(base, TPT)], pid_v)
    pltpu.sync_copy(wa_hbm.at[pl.ds(base, TPT)], wa_v)
    pltpu.sync_copy(wb_hbm.at[pl.ds(base, TPT)], wb_v)
    pltpu.sync_copy(start_hbm.at[wid], start_v)

    for k in range(NBUK // L):
        cnt_v[pl.ds(k * L, L)] = jnp.zeros((L,), jnp.int32)

    for ci in range(TPT // XCH):
        for k in range(XCH // L):
            j = ci * XCH + k * L
            p = pid_v[pl.ds(j, L)]
            sk, sv = plsc.sort_key_val(p, iota)
            tmp_v[...] = sk
            prevk = plsc.load_gather(tmp_v, [jnp.maximum(iota - 1, 0)])
            nextk = plsc.load_gather(tmp_v, [jnp.minimum(iota + 1, L - 1)])
            is_start = (iota == 0) | (sk != prevk)
            is_end = (iota == L - 1) | (sk != nextk)
            lb = plsc.cummax(jnp.where(is_start, iota, 0))
            rank = iota - lb
            cbase = plsc.load_gather(cnt_v, [sk])
            sbase = plsc.load_gather(start_v, [sk])
            slot = sbase + cbase + rank
            plsc.store_scatter(cnt_v, [sk], cbase + rank + 1, mask=is_end)
            plsc.store_scatter(tmp_v, [sv], slot)
            slot_t = tmp_v[...]
            pos_v[pl.ds(j, L)] = slot_t
            idx_v[pl.ds(k * L, L)] = slot_t
            was_v[pl.ds(k * L, L)] = wa_v[pl.ds(j, L)]
            wbs_v[pl.ds(k * L, L)] = wb_v[pl.ds(j, L)]
        pltpu.sync_copy(x_hbm.at[pl.ds(base + ci * XCH, XCH)], xrow_v)
        pltpu.async_copy(xrow_v, xg_hbm.at[idx_v], sem).wait()
        pltpu.async_copy(was_v, swa_hbm.at[idx_v], sem).wait()
        pltpu.async_copy(wbs_v, swb_hbm.at[idx_v], sem).wait()

    pltpu.sync_copy(pos_v, pos_hbm.at[pl.ds(base, TPT)])


def _route(x, pid, wa, wb, start_all):
    mesh = plsc.VectorSubcoreMesh(core_axis_name="c", subcore_axis_name="s")
    f = pl.kernel(
        _route_body,
        out_type=[
            jax.ShapeDtypeStruct((CAPP, D), jnp.float32),
            jax.ShapeDtypeStruct((CAPP,), jnp.float32),
            jax.ShapeDtypeStruct((CAPP,), jnp.float32),
            jax.ShapeDtypeStruct((B,), jnp.int32),
        ],
        mesh=mesh,
        scratch_types=[
            pltpu.VMEM((TPT,), jnp.int32),
            pltpu.VMEM((TPT,), jnp.float32),
            pltpu.VMEM((TPT,), jnp.float32),
            pltpu.VMEM((NBUK,), jnp.int32),
            pltpu.VMEM((NBUK,), jnp.int32),
            pltpu.VMEM((TPT,), jnp.int32),
            pltpu.VMEM((XCH, D), jnp.float32),
            pltpu.VMEM((XCH,), jnp.int32),
            pltpu.VMEM((XCH,), jnp.float32),
            pltpu.VMEM((XCH,), jnp.float32),
            pltpu.VMEM((L,), jnp.int32),
            pltpu.SemaphoreType.DMA,
        ],
        compiler_params=pltpu.CompilerParams(needs_layout_passes=False),
    )
    return f(x, pid, wa, wb, start_all)


def _mm_body(bea_ref, beb_ref, vcnt_ref, xg_ref, swa_ref, swb_ref,
             we_ref, be_ref, z_ref):
    i = pl.program_id(0)
    ea = bea_ref[i]
    eb = beb_ref[i]
    xgb = xg_ref[...]
    ya = lax.dot_general(xgb, we_ref[ea], (((1,), (0,)), ((), ())),
                         preferred_element_type=jnp.float32)
    yb = lax.dot_general(xgb, we_ref[eb], (((1,), (0,)), ((), ())),
                         preferred_element_type=jnp.float32)
    wa = swa_ref[...][:, None]
    wb = swb_ref[...][:, None]
    z = wa * (ya + be_ref[ea][None, :]) + wb * (yb + be_ref[eb][None, :])
    row = lax.broadcasted_iota(jnp.int32, (GP, O), 0)
    z_ref[...] = jnp.where(row < vcnt_ref[i], z, 0.0)


def _mm(bea, beb, vcnt, xg, swa, swb, We, be):
    grid_spec = pltpu.PrefetchScalarGridSpec(
        num_scalar_prefetch=3,
        grid=(NBLK,),
        in_specs=[
            pl.BlockSpec((GP, D), lambda i, *_: (i, 0)),
            pl.BlockSpec((GP,), lambda i, *_: (i,)),
            pl.BlockSpec((GP,), lambda i, *_: (i,)),
            pl.BlockSpec((E, D, O), lambda i, *_: (0, 0, 0)),
            pl.BlockSpec((E, O), lambda i, *_: (0, 0)),
        ],
        out_specs=pl.BlockSpec((GP, O), lambda i, *_: (i, 0)),
    )
    return pl.pallas_call(
        _mm_body,
        grid_spec=grid_spec,
        out_shape=jax.ShapeDtypeStruct((CAPP, O), jnp.float32),
    )(bea, beb, vcnt, xg, swa, swb, We, be)


def _perm_body(z_hbm, pos_hbm, out_hbm, pidx_v, rows_v, sem):
    c = lax.axis_index("c")
    s = lax.axis_index("s")
    wid = s * NC + c
    base = wid * TPT
    for ci in range(TPT // XCH):
        pltpu.sync_copy(pos_hbm.at[pl.ds(base + ci * XCH, XCH)], pidx_v)
        pltpu.async_copy(z_hbm.at[pidx_v], rows_v, sem).wait()
        pltpu.sync_copy(rows_v, out_hbm.at[pl.ds(base + ci * XCH, XCH)])


def _perm(z, pos):
    mesh = plsc.VectorSubcoreMesh(core_axis_name="c", subcore_axis_name="s")
    f = pl.kernel(
        _perm_body,
        out_type=jax.ShapeDtypeStruct((B, O), jnp.float32),
        mesh=mesh,
        scratch_types=[
            pltpu.VMEM((XCH,), jnp.int32),
            pltpu.VMEM((XCH, D), jnp.float32),
            pltpu.SemaphoreType.DMA,
        ],
    )
    return f(z, pos)


def kernel(insample_y, Wg, bg, We, be):
    pid, wa, wb, cnt_blocks = _gate(insample_y, Wg, bg)
    counts_all = cnt_blocks.reshape(NW, NBUK)
    counts = jnp.sum(counts_all, axis=0)
    padded = ((counts + GP - 1) // GP) * GP
    ends = jnp.cumsum(padded)
    ps = ends - padded
    prefix = jnp.cumsum(counts_all, axis=0) - counts_all
    start_all = (ps[None, :] + prefix).astype(jnp.int32)

    xg, swa, swb, pos = _route(insample_y, pid, wa, wb, start_all)

    blk = jnp.arange(NBLK, dtype=jnp.int32) * GP
    bk = jnp.minimum(jnp.searchsorted(ends, blk, side="right"),
                     NBUK - 1).astype(jnp.int32)
    bea = bk // E
    beb = bk % E
    vcnt = jnp.clip(counts[bk] - (blk - ps[bk]), 0, GP).astype(jnp.int32)

    z = _mm(bea, beb, vcnt, xg, swa, swb, We, be)
    return _perm(z, pos)
